# Initial kernel scaffold; baseline (speedup 1.0000x reference)
import jax
import jax.numpy as jnp
from jax.experimental import pallas as pl


def kernel(x, table):
    raise NotImplementedError("write your pallas kernel here")


# speedup vs baseline: 9.2910x; 9.2910x over previous
import functools

import jax
import jax.numpy as jnp
from jax import lax
from jax.experimental import pallas as pl
from jax.experimental.pallas import tpu as pltpu
from jax.experimental.pallas import tpu_sc as plsc

VOCAB = 100000
D_MODEL = 128

_info = plsc.get_sparse_core_info()
_NC, _NS = _info.num_cores, _info.num_subcores
_NW = _NC * _NS

_CHUNK = 128


@functools.partial(jax.jit, static_argnames=("b_per_w",))
def _gather_sc(x_flat, table, *, b_per_w):
    n_chunks = b_per_w // _CHUNK
    B = _NW * b_per_w
    mesh = plsc.VectorSubcoreMesh(core_axis_name="c", subcore_axis_name="s")

    @functools.partial(
        pl.kernel,
        mesh=mesh,
        out_type=jax.ShapeDtypeStruct((B, D_MODEL), jnp.float32),
        scratch_types=[
            pltpu.VMEM((n_chunks, _CHUNK), jnp.int32),
            pltpu.VMEM((2, _CHUNK, D_MODEL), jnp.float32),
            pltpu.SemaphoreType.DMA,
            pltpu.SemaphoreType.DMA,
            pltpu.SemaphoreType.DMA,
        ],
    )
    def k(x_hbm, table_hbm, out_hbm, idx_v, rows_v, gsem, osem, isem):
        wid = lax.axis_index("s") * _NC + lax.axis_index("c")
        base = wid * b_per_w

        staged = pltpu.make_async_copy(x_hbm.at[wid], idx_v.at[...], isem)
        staged.start()
        staged.wait()

        def gather(j, slot):
            return pltpu.make_async_copy(
                table_hbm.at[idx_v.at[j]],
                rows_v.at[slot],
                gsem,
            )

        def store(j, slot):
            return pltpu.make_async_copy(
                rows_v.at[slot],
                out_hbm.at[pl.ds(base + j * _CHUNK, _CHUNK)],
                osem,
            )

        gather(0, 0).start()

        def body(j, _):
            slot = lax.rem(j, 2)
            nslot = lax.rem(j + 1, 2)

            @pl.when(j > 0)
            def _():
                store(j - 1, nslot).wait()

            @pl.when(j + 1 < n_chunks)
            def _():
                gather(j + 1, nslot).start()

            gather(j, slot).wait()
            store(j, slot).start()
            return 0

        lax.fori_loop(0, n_chunks, body, 0)
        store(n_chunks - 1, lax.rem(n_chunks - 1, 2)).wait()

    return k(x_flat, table)


def kernel(x, table):
    B_total = x.shape[0] * x.shape[1]
    x_flat = jnp.reshape(x.astype(jnp.int32), (_NW, B_total // (_NW * _CHUNK), _CHUNK))
    b_per_w = B_total // _NW
    out = _gather_sc(x_flat, table, b_per_w=b_per_w)
    return jnp.reshape(out, (x.shape[0], x.shape[1], D_MODEL))
